# baseline (speedup 1.0000x reference)
import functools

import jax
import jax.numpy as jnp
from jax import lax
from jax.experimental import pallas as pl
from jax.experimental.pallas import tpu as pltpu
from jax.experimental.pallas import tpu_sc as plsc

LANES = 16
NCORES = 2
NSUB = 16
NW = NCORES * NSUB
CHUNK = 128
H = 128

N_N = 10000
M_M = 5000
E_E = 320000
RT = 80
ROWS = RT * NW
EP = ROWS * CHUNK
NP_ = 10112
MP_ = 5120
NPT = NP_ // NSUB
MPT = MP_ // NSUB
MRANGE = MP_ // NW
QCAP = 16384
DSENT = 1 << 20


def _chunks(total, step=CHUNK):
    out, off = [], 0
    while off < total:
        b = min(step, total - off)
        out.append((off, b))
        off += b
    return out


def _mm_kernel(x_ref, w_ref, b_ref, o_ref, *, act):
    y = jnp.dot(x_ref[...], w_ref[...], preferred_element_type=jnp.float32)
    y = y + b_ref[...]
    if act:
        y = jnp.tanh(y)
    o_ref[...] = y


def _mm(x, w, b, act=True, block=1024):
    n, k = x.shape
    m = w.shape[1]
    b2 = b.reshape(1, m)
    return pl.pallas_call(
        functools.partial(_mm_kernel, act=act),
        grid=(pl.cdiv(n, block),),
        in_specs=[
            pl.BlockSpec((block, k), lambda i: (i, 0)),
            pl.BlockSpec((k, m), lambda i: (0, 0)),
            pl.BlockSpec((1, m), lambda i: (0, 0)),
        ],
        out_specs=pl.BlockSpec((block, m), lambda i: (i, 0)),
        out_shape=jax.ShapeDtypeStruct((n, m), jnp.float32),
    )(x, w, b2)


def _bn(x, g, b):
    mu = jnp.mean(x, axis=0, keepdims=True)
    var = jnp.var(x, axis=0, keepdims=True)
    return (x - mu) / jnp.sqrt(var + 1e-5) * g + b


def _mesh():
    return plsc.VectorSubcoreMesh(core_axis_name="c", subcore_axis_name="s")


def _zero_vec_loop(ref, n16, value=0.0):
    def body(i, _):
        ref[pl.ds(i * LANES, LANES)] = jnp.full((LANES,), value, jnp.float32)
        return 0
    lax.fori_loop(0, n16, body, 0)


def _make_counts():
    @functools.partial(
        pl.kernel,
        mesh=_mesh(),
        compiler_params=pltpu.CompilerParams(needs_layout_passes=False),
        out_type=[
            jax.ShapeDtypeStruct((NW * NP_,), jnp.float32),
            jax.ShapeDtypeStruct((NW * MP_,), jnp.float32),
        ],
        scratch_types=[
            pltpu.VMEM((RT, CHUNK), jnp.int32),
            pltpu.VMEM((RT, CHUNK), jnp.int32),
            pltpu.VMEM((NP_,), jnp.float32),
            pltpu.VMEM((MP_,), jnp.float32),
        ],
    )
    def counts(srcs, dsts, ocs, ocd, sv, dv, cs, cd):
        c = lax.axis_index("c")
        s = lax.axis_index("s")
        wid = c * NSUB + s
        _zero_vec_loop(cs, NP_ // LANES)
        _zero_vec_loop(cd, MP_ // LANES)
        pltpu.sync_copy(srcs.at[pl.ds(wid * RT, RT)], sv)
        pltpu.sync_copy(dsts.at[pl.ds(wid * RT, RT)], dv)
        ones = jnp.ones((LANES,), jnp.float32)

        @plsc.parallel_loop(0, RT, unroll=4)
        def _body(i):
            for g in range(CHUNK // LANES):
                dsl = pl.ds(g * LANES, LANES)
                plsc.addupdate_scatter(cs, [sv[i, dsl]], ones)
                plsc.addupdate_scatter(cd, [dv[i, dsl]], ones)
        pltpu.sync_copy(cs, ocs.at[pl.ds(wid * NP_, NP_)])
        pltpu.sync_copy(cd, ocd.at[pl.ds(wid * MP_, MP_)])

    return counts


def _make_rowsum(Sp, weighted):
    rpt = Sp // NSUB
    BB = 8

    scratch = [
        pltpu.VMEM((BB, CHUNK), jnp.int32),
        pltpu.VMEM((BB, CHUNK), jnp.int32),
        pltpu.VMEM((BB, CHUNK), jnp.float32),
        pltpu.VMEM((CHUNK, H), jnp.float32),
        pltpu.VMEM((CHUNK, H), jnp.float32),
        pltpu.VMEM_SHARED((Sp, H), jnp.float32),
        pltpu.SemaphoreType.DMA,
        pltpu.SemaphoreType.DMA,
        pltpu.SemaphoreType.DMA,
        pltpu.SemaphoreType.DMA,
    ]

    @functools.partial(
        pl.kernel,
        mesh=_mesh(),
        compiler_params=pltpu.CompilerParams(needs_layout_passes=False),
        out_type=jax.ShapeDtypeStruct((NCORES, Sp, H), jnp.float32),
        scratch_types=scratch,
    )
    def rowsum(table, gidx, sidx, ew, out,
               gb, sb, eb, rows0, rows1, acc, sem0, sem1, ssem0, ssem1):
        c = lax.axis_index("c")
        s = lax.axis_index("s")
        wid = c * NSUB + s

        def zb(i, _):
            rows0[i // (H // LANES),
                  pl.ds((i % (H // LANES)) * LANES, LANES)] = jnp.zeros(
                (LANES,), jnp.float32)
            return 0
        lax.fori_loop(0, CHUNK * (H // LANES), zb, 0)

        base = s * rpt
        for off, blk in _chunks(rpt, CHUNK):
            pltpu.sync_copy(rows0.at[pl.ds(0, blk)],
                            acc.at[pl.ds(base + off, blk)])
        plsc.subcore_barrier()

        bufs = (rows0, rows1)
        sems = (sem0, sem1)
        ssems = (ssem0, ssem1)

        def scale_buf(j, buf):
            def scale(g, _):
                e16 = eb[j, pl.ds(g * LANES, LANES)]
                for lane in range(LANES):
                    e = g * LANES + lane
                    f = lax.broadcast_in_dim(e16[lane], (LANES,), ())
                    for jj in range(H // LANES):
                        dj = pl.ds(jj * LANES, LANES)
                        buf[e, dj] = buf[e, dj] * f
                return 0
            lax.fori_loop(0, CHUNK // LANES, scale, 0)

        def body(b, _):
            r0 = wid * RT + b * BB
            pltpu.sync_copy(gidx.at[pl.ds(r0, BB)], gb)
            pltpu.sync_copy(sidx.at[pl.ds(r0, BB)], sb)
            if weighted:
                pltpu.sync_copy(ew.at[pl.ds(r0, BB)], eb)
            gds = [None, None]
            sds = [None, None]
            gds[0] = pltpu.async_copy(table.at[gb.at[0]], bufs[0], sems[0])
            for j in range(BB):
                if j + 1 < BB:
                    if j >= 1:
                        sds[(j - 1) % 2].wait()
                    gds[(j + 1) % 2] = pltpu.async_copy(
                        table.at[gb.at[j + 1]], bufs[(j + 1) % 2],
                        sems[(j + 1) % 2])
                gds[j % 2].wait()
                buf = bufs[j % 2]
                if weighted:
                    scale_buf(j, buf)
                sds[j % 2] = pltpu.async_copy(
                    buf, acc.at[sb.at[j]], ssems[j % 2], add=True)
            sds[0].wait()
            sds[1].wait()
            return 0

        lax.fori_loop(0, RT // BB, body, 0)
        plsc.subcore_barrier()
        for off, blk in _chunks(rpt, CHUNK):
            pltpu.sync_copy(acc.at[pl.ds(base + off, blk)],
                            out.at[c, pl.ds(base + off, blk)])

    return rowsum


def _make_zsums(xside):
    Sp = NP_ if xside else MP_

    @functools.partial(
        pl.kernel,
        mesh=_mesh(),
        compiler_params=pltpu.CompilerParams(needs_layout_passes=False),
        out_type=[
            jax.ShapeDtypeStruct((NW * Sp,), jnp.float32),
            jax.ShapeDtypeStruct((ROWS, CHUNK), jnp.float32),
        ],
        scratch_types=[
            pltpu.VMEM((N_N,), jnp.float32),
            pltpu.VMEM((M_M,), jnp.float32),
            pltpu.VMEM((LANES,), jnp.float32),
            pltpu.VMEM((RT, CHUNK), jnp.int32),
            pltpu.VMEM((RT, CHUNK), jnp.int32),
            pltpu.VMEM((RT, CHUNK), jnp.int32),
            pltpu.VMEM((Sp,), jnp.float32),
            pltpu.VMEM((RT, CHUNK), jnp.float32),
        ],
    )
    def zsums(sn, se, cm, srcg, dstg, sidx, oz, oew,
              snv, sev, cmv, sgv, dgv, siv, zl, eav):
        c = lax.axis_index("c")
        s = lax.axis_index("s")
        wid = c * NSUB + s
        pltpu.sync_copy(sn, snv)
        pltpu.sync_copy(se, sev)
        pltpu.sync_copy(cm, cmv)
        pltpu.sync_copy(srcg.at[pl.ds(wid * RT, RT)], sgv)
        pltpu.sync_copy(dstg.at[pl.ds(wid * RT, RT)], dgv)
        pltpu.sync_copy(sidx.at[pl.ds(wid * RT, RT)], siv)
        _zero_vec_loop(zl, Sp // LANES)
        cmax = cmv[...]

        @plsc.parallel_loop(0, RT, unroll=4)
        def _body(i):
            for g in range(CHUNK // LANES):
                dsl = pl.ds(g * LANES, LANES)
                a = plsc.load_gather(snv, [sgv[i, dsl]])
                b = plsc.load_gather(sev, [dgv[i, dsl]])
                t = a + b
                sc = jnp.where(t >= 0, t, 0.2 * t)
                mb = (a if xside else b) + cmax
                mb = jnp.where(mb >= 0, mb, 0.2 * mb)
                ew = jnp.exp(sc - mb)
                eav[i, dsl] = ew
                plsc.addupdate_scatter(zl, [siv[i, dsl]], ew)
        pltpu.sync_copy(zl, oz.at[pl.ds(wid * Sp, Sp)])
        pltpu.sync_copy(eav, oew.at[pl.ds(wid * RT, RT)])

    return zsums


def _make_segmin():
    SBLK = 64
    nblk = ROWS // SBLK

    @functools.partial(
        pl.kernel,
        mesh=_mesh(),
        compiler_params=pltpu.CompilerParams(needs_layout_passes=False),
        out_type=jax.ShapeDtypeStruct((MP_, H), jnp.float32),
        scratch_types=[
            pltpu.VMEM((64, CHUNK), jnp.int32),
            pltpu.VMEM((64, CHUNK), jnp.int32),
            pltpu.VMEM((QCAP,), jnp.int32),
            pltpu.VMEM((QCAP,), jnp.int32),
            pltpu.VMEM((CHUNK, H), jnp.float32),
            pltpu.VMEM((MRANGE + 1, H), jnp.float32),
            pltpu.SemaphoreType.DMA,
        ],
    )
    def segmin(xn3, srcg, dstm, out, sgb, dmb, qsrc, qloc, rows, acc, sem):
        c = lax.axis_index("c")
        s = lax.axis_index("s")
        wid = c * NSUB + s
        lo = wid * MRANGE

        def ini(i, _):
            acc[i // 8, pl.ds((i % 8) * LANES, LANES)] = jnp.full(
                (LANES,), jnp.inf, jnp.float32)
            return 0
        lax.fori_loop(0, (MRANGE + 1) * 8, ini, 0)

        def iniq(i, _):
            dsl = pl.ds(i * LANES, LANES)
            qsrc[dsl] = jnp.zeros((LANES,), jnp.int32)
            qloc[dsl] = jnp.full((LANES,), MRANGE, jnp.int32)
            return 0
        lax.fori_loop(0, QCAP // LANES, iniq, 0)

        def blk(b, qpos):
            pltpu.sync_copy(srcg.at[pl.ds(b * SBLK, SBLK)], sgb)
            pltpu.sync_copy(dstm.at[pl.ds(b * SBLK, SBLK)], dmb)

            def row(i, qp):
                for g in range(CHUNK // LANES):
                    dsl = pl.ds(g * LANES, LANES)
                    d16 = dmb[i, dsl]
                    s16 = sgb[i, dsl]
                    m = (d16 >= lo) & (d16 < lo + MRANGE)
                    cnt = jnp.sum(m.astype(jnp.int32))
                    qp = jnp.minimum(qp, QCAP - LANES)
                    plsc.store_compressed(qsrc.at[pl.ds(qp, LANES)], s16,
                                          mask=m)
                    plsc.store_compressed(qloc.at[pl.ds(qp, LANES)],
                                          d16 - lo, mask=m)
                    qp = qp + cnt
                return qp

            return lax.fori_loop(0, SBLK, row, qpos)

        qpos = lax.fori_loop(0, nblk, blk, 0)
        nb = (qpos + CHUNK - 1) // CHUNK

        def bat(b, _):
            pltpu.async_copy(xn3.at[qsrc.at[pl.ds(b * CHUNK, CHUNK)]],
                             rows, sem).wait()

            def per_grp(g, _):
                ql16 = qloc[pl.ds(b * CHUNK + g * LANES, LANES)]
                for lane in range(LANES):
                    e = g * LANES + lane
                    dl = ql16[lane]
                    for j in range(H // LANES):
                        dj = pl.ds(j * LANES, LANES)
                        acc[dl, dj] = jnp.minimum(acc[dl, dj], rows[e, dj])
                return 0

            lax.fori_loop(0, CHUNK // LANES, per_grp, 0)
            return 0

        lax.fori_loop(0, nb, bat, 0)
        pltpu.sync_copy(acc.at[pl.ds(0, MRANGE)], out.at[pl.ds(lo, MRANGE)])

    return segmin


def kernel(x, x_struct, x_e, edge_index, W_her, b_her, W_sfr, b_sfr, Wn1, We1, an1, ae1, Wn2, We2, an2, ae2, Wn3, We3, an3, ae3, gamma1, beta1, W_fuse, b_fuse, gamma2, beta2, W_c1, b_c1, W_c2, b_c2):
    src, dst = edge_index[0], edge_index[1]
    N = x.shape[0]
    M = x_e.shape[0]
    E = src.shape[0]
    pad = EP - E

    i32 = jnp.int32
    ew_dummy = jnp.zeros((ROWS, CHUNK), jnp.float32)
    srcg = jnp.concatenate([src, jnp.zeros((pad,), i32)]).reshape(ROWS, CHUNK)
    dstg = jnp.concatenate([dst, jnp.zeros((pad,), i32)]).reshape(ROWS, CHUNK)
    srcs = jnp.concatenate(
        [src, jnp.full((pad,), NP_ - 1, i32)]).reshape(ROWS, CHUNK)
    dsts = jnp.concatenate(
        [dst, jnp.full((pad,), MP_ - 1, i32)]).reshape(ROWS, CHUNK)
    dstm = jnp.concatenate(
        [dst, jnp.full((pad,), DSENT, i32)]).reshape(ROWS, CHUNK)

    USE_COUNTS, USE_ROWSUM, USE_LAYER, USE_MIN = True, True, True, True
    counts_k = _make_counts()
    rowsum_m = _make_rowsum(MP_, False)
    rowsum_n = _make_rowsum(NP_, False)
    wrowsum_n = _make_rowsum(NP_, True)
    wrowsum_m = _make_rowsum(MP_, True)
    zsums_x = _make_zsums(True)
    zsums_e = _make_zsums(False)
    segmin_k = _make_segmin()

    xe = _mm(x_e, W_her, b_her, act=True)
    xn0 = _mm(x, W_sfr, b_sfr, act=True)

    if USE_COUNTS:
        cs_p, cd_p = counts_k(srcs, dsts)
        cnt_src = cs_p.reshape(NW, NP_).sum(axis=0)[:N]
        cnt_dst = cd_p.reshape(NW, MP_).sum(axis=0)[:M]
    else:
        cnt_src = jax.ops.segment_sum(jnp.ones((E,)), src, num_segments=N)
        cnt_dst = jax.ops.segment_sum(jnp.ones((E,)), dst, num_segments=M)

    if USE_ROWSUM:
        he_p = rowsum_m(xn0, srcg, dsts, ew_dummy)
        he_s = (he_p[0] + he_p[1])[:M]
    else:
        he_s = jax.ops.segment_sum(xn0[src], dst, num_segments=M)
    he = he_s / jnp.maximum(cnt_dst, 1.0)[:, None]
    if USE_ROWSUM:
        bk_p = rowsum_n(he, dstg, srcs, ew_dummy)
        bk_s = (bk_p[0] + bk_p[1])[:N]
    else:
        bk_s = jax.ops.segment_sum(he[dst], src, num_segments=N)
    xn = jnp.tanh(xn0 + bk_s / jnp.maximum(cnt_src, 1.0)[:, None])

    zero_b = jnp.zeros((H,), jnp.float32)
    for Wn, We, an, ae in ((Wn1, We1, an1, ae1), (Wn2, We2, an2, ae2),
                           (Wn3, We3, an3, ae3)):
        qn = _mm(xn, Wn, zero_b, act=False)
        qe = _mm(xe, We, zero_b, act=False)
        sn = qn @ an
        se = qe @ ae
        if USE_LAYER:
            cse = jnp.full((LANES,), se.max(), jnp.float32)
            csn = jnp.full((LANES,), sn.max(), jnp.float32)
            za, ea2 = zsums_x(sn, se, cse, srcg, dstg, srcs)
            zb, eb2 = zsums_e(sn, se, csn, srcg, dstg, dsts)
            DIAG_XLA_ROWS = True
            if DIAG_XLA_ROWS:
                eaf = ea2.reshape(-1)[:E]
                ebf = eb2.reshape(-1)[:E]
                uj = jax.ops.segment_sum(eaf[:, None] * qe[dst], src,
                                         num_segments=N)
                vj = jax.ops.segment_sum(ebf[:, None] * qn[src], dst,
                                         num_segments=M)
                xacc = jnp.stack([uj, jnp.zeros_like(uj)])
                eacc = jnp.stack([vj, jnp.zeros_like(vj)])
                xacc = jnp.pad(xacc, ((0, 0), (0, NP_ - N), (0, 0)))
                eacc = jnp.pad(eacc, ((0, 0), (0, MP_ - M), (0, 0)))
            else:
                xacc = wrowsum_n(qe, dstg, srcs, ea2)
                eacc = wrowsum_m(qn, srcg, dsts, eb2)
            u = (xacc[0] + xacc[1])[:N]
            v = (eacc[0] + eacc[1])[:M]
            zan = za.reshape(NW, NP_).sum(axis=0)[:N]
            zbm = zb.reshape(NW, MP_).sum(axis=0)[:M]
        else:
            mhn = jax.nn.leaky_relu(sn + se.max(), 0.2)
            mhe = jax.nn.leaky_relu(se + sn.max(), 0.2)
            score = jax.nn.leaky_relu(sn[src] + se[dst], 0.2)
            ea = jnp.exp(score - mhn[src])
            eb = jnp.exp(score - mhe[dst])
            u = jax.ops.segment_sum(ea[:, None] * qe[dst], src, num_segments=N)
            v = jax.ops.segment_sum(eb[:, None] * qn[src], dst, num_segments=M)
            zan = jax.ops.segment_sum(ea, src, num_segments=N)
            zbm = jax.ops.segment_sum(eb, dst, num_segments=M)
        xn = jnp.tanh(u / (zan[:, None] + 1e-16) + xn)
        xe = jnp.tanh(v / (zbm[:, None] + 1e-16) + xe)

    if USE_MIN:
        agg = segmin_k(xn, srcg, dstm)[:M]
    else:
        agg = jax.ops.segment_min(xn[src], dst, num_segments=M)
    h = jnp.concatenate([agg, xe], axis=1)
    h = _bn(h, gamma1, beta1)
    h = jnp.tanh(_mm(h, W_fuse, b_fuse, act=False))
    h = _bn(h, gamma2, beta2)
    h = jnp.tanh(_mm(h, W_c1, b_c1, act=False))
    return _mm(h, W_c2, b_c2, act=False)


# speedup vs baseline: 1.9225x; 1.9225x over previous
import functools

import jax
import jax.numpy as jnp
from jax import lax
from jax.experimental import pallas as pl
from jax.experimental.pallas import tpu as pltpu
from jax.experimental.pallas import tpu_sc as plsc

LANES = 16
NCORES = 2
NSUB = 16
NW = NCORES * NSUB
CHUNK = 128
H = 128

N_N = 10000
M_M = 5000
E_E = 320000
RT = 80
ROWS = RT * NW
EP = ROWS * CHUNK
NP_ = 10112
MP_ = 5120
NPT = NP_ // NSUB
MPT = MP_ // NSUB
MRANGE = MP_ // NW
QCAP = 16384
DSENT = 1 << 20


def _chunks(total, step=CHUNK):
    out, off = [], 0
    while off < total:
        b = min(step, total - off)
        out.append((off, b))
        off += b
    return out


def _mm_kernel(x_ref, w_ref, b_ref, o_ref, *, act):
    y = jnp.dot(x_ref[...], w_ref[...], preferred_element_type=jnp.float32)
    y = y + b_ref[...]
    if act:
        y = jnp.tanh(y)
    o_ref[...] = y


def _mm(x, w, b, act=True, block=1024):
    n, k = x.shape
    m = w.shape[1]
    b2 = b.reshape(1, m)
    return pl.pallas_call(
        functools.partial(_mm_kernel, act=act),
        grid=(pl.cdiv(n, block),),
        in_specs=[
            pl.BlockSpec((block, k), lambda i: (i, 0)),
            pl.BlockSpec((k, m), lambda i: (0, 0)),
            pl.BlockSpec((1, m), lambda i: (0, 0)),
        ],
        out_specs=pl.BlockSpec((block, m), lambda i: (i, 0)),
        out_shape=jax.ShapeDtypeStruct((n, m), jnp.float32),
    )(x, w, b2)


def _bn(x, g, b):
    mu = jnp.mean(x, axis=0, keepdims=True)
    var = jnp.var(x, axis=0, keepdims=True)
    return (x - mu) / jnp.sqrt(var + 1e-5) * g + b


def _mesh():
    return plsc.VectorSubcoreMesh(core_axis_name="c", subcore_axis_name="s")


def _zero_vec_loop(ref, n16, value=0.0):
    def body(i, _):
        ref[pl.ds(i * LANES, LANES)] = jnp.full((LANES,), value, jnp.float32)
        return 0
    lax.fori_loop(0, n16, body, 0)


def _make_counts():
    @functools.partial(
        pl.kernel,
        mesh=_mesh(),
        compiler_params=pltpu.CompilerParams(needs_layout_passes=False),
        out_type=[
            jax.ShapeDtypeStruct((NW * NP_,), jnp.float32),
            jax.ShapeDtypeStruct((NW * MP_,), jnp.float32),
        ],
        scratch_types=[
            pltpu.VMEM((RT, CHUNK), jnp.int32),
            pltpu.VMEM((RT, CHUNK), jnp.int32),
            pltpu.VMEM((NP_,), jnp.float32),
            pltpu.VMEM((MP_,), jnp.float32),
        ],
    )
    def counts(srcs, dsts, ocs, ocd, sv, dv, cs, cd):
        c = lax.axis_index("c")
        s = lax.axis_index("s")
        wid = c * NSUB + s
        _zero_vec_loop(cs, NP_ // LANES)
        _zero_vec_loop(cd, MP_ // LANES)
        pltpu.sync_copy(srcs.at[pl.ds(wid * RT, RT)], sv)
        pltpu.sync_copy(dsts.at[pl.ds(wid * RT, RT)], dv)
        ones = jnp.ones((LANES,), jnp.float32)

        @plsc.parallel_loop(0, RT, unroll=4)
        def _body(i):
            for g in range(CHUNK // LANES):
                dsl = pl.ds(g * LANES, LANES)
                plsc.addupdate_scatter(cs, [sv[i, dsl]], ones)
                plsc.addupdate_scatter(cd, [dv[i, dsl]], ones)
        pltpu.sync_copy(cs, ocs.at[pl.ds(wid * NP_, NP_)])
        pltpu.sync_copy(cd, ocd.at[pl.ds(wid * MP_, MP_)])

    return counts


def _make_rowsum(Sp, weighted):
    rpt = Sp // NSUB
    BB = 16

    scratch = [
        pltpu.VMEM((BB, CHUNK), jnp.int32),
        pltpu.VMEM((BB, CHUNK), jnp.int32),
        pltpu.VMEM((BB, CHUNK), jnp.float32),
        pltpu.VMEM((CHUNK, H), jnp.float32),
        pltpu.VMEM((CHUNK, H), jnp.float32),
        pltpu.VMEM_SHARED((Sp, H), jnp.float32),
        pltpu.SemaphoreType.DMA,
        pltpu.SemaphoreType.DMA,
        pltpu.SemaphoreType.DMA,
        pltpu.SemaphoreType.DMA,
    ]

    @functools.partial(
        pl.kernel,
        mesh=_mesh(),
        compiler_params=pltpu.CompilerParams(needs_layout_passes=False),
        out_type=jax.ShapeDtypeStruct((NCORES, Sp, H), jnp.float32),
        scratch_types=scratch,
    )
    def rowsum(table, gidx, sidx, ew, out,
               gb, sb, eb, rows0, rows1, acc, sem0, sem1, ssem0, ssem1):
        c = lax.axis_index("c")
        s = lax.axis_index("s")
        wid = c * NSUB + s

        def zb(i, _):
            rows0[i // (H // LANES),
                  pl.ds((i % (H // LANES)) * LANES, LANES)] = jnp.zeros(
                (LANES,), jnp.float32)
            return 0
        lax.fori_loop(0, CHUNK * (H // LANES), zb, 0)

        base = s * rpt
        for off, blk in _chunks(rpt, CHUNK):
            pltpu.sync_copy(rows0.at[pl.ds(0, blk)],
                            acc.at[pl.ds(base + off, blk)])
        plsc.subcore_barrier()

        bufs = (rows0, rows1)
        sems = (sem0, sem1)
        ssems = (ssem0, ssem1)

        def scale_buf(j, buf):
            def scale(g, _):
                e16 = eb[j, pl.ds(g * LANES, LANES)]
                for lane in range(LANES):
                    e = g * LANES + lane
                    f = lax.broadcast_in_dim(e16[lane], (LANES,), ())
                    for jj in range(H // LANES):
                        dj = pl.ds(jj * LANES, LANES)
                        buf[e, dj] = buf[e, dj] * f
                return 0
            lax.fori_loop(0, CHUNK // LANES, scale, 0)

        def body(b, _):
            r0 = wid * RT + b * BB
            pltpu.sync_copy(gidx.at[pl.ds(r0, BB)], gb)
            pltpu.sync_copy(sidx.at[pl.ds(r0, BB)], sb)
            if weighted:
                pltpu.sync_copy(ew.at[pl.ds(r0, BB)], eb)
            gds = [None, None]
            sds = [None, None]
            gds[0] = pltpu.async_copy(table.at[gb.at[0]], bufs[0], sems[0])
            for j in range(BB):
                if j + 1 < BB:
                    if j >= 1:
                        sds[(j - 1) % 2].wait()
                    gds[(j + 1) % 2] = pltpu.async_copy(
                        table.at[gb.at[j + 1]], bufs[(j + 1) % 2],
                        sems[(j + 1) % 2])
                gds[j % 2].wait()
                buf = bufs[j % 2]
                if weighted:
                    scale_buf(j, buf)
                sds[j % 2] = pltpu.async_copy(
                    buf, acc.at[sb.at[j]], ssems[j % 2], add=True)
            sds[0].wait()
            sds[1].wait()
            return 0

        lax.fori_loop(0, RT // BB, body, 0)
        plsc.subcore_barrier()
        for off, blk in _chunks(rpt, CHUNK):
            pltpu.sync_copy(acc.at[pl.ds(base + off, blk)],
                            out.at[c, pl.ds(base + off, blk)])

    return rowsum


def _make_zsums(xside):
    Sp = NP_ if xside else MP_

    @functools.partial(
        pl.kernel,
        mesh=_mesh(),
        compiler_params=pltpu.CompilerParams(needs_layout_passes=False),
        out_type=[
            jax.ShapeDtypeStruct((NW * Sp,), jnp.float32),
            jax.ShapeDtypeStruct((ROWS, CHUNK), jnp.float32),
        ],
        scratch_types=[
            pltpu.VMEM((N_N,), jnp.float32),
            pltpu.VMEM((M_M,), jnp.float32),
            pltpu.VMEM((LANES,), jnp.float32),
            pltpu.VMEM((RT, CHUNK), jnp.int32),
            pltpu.VMEM((RT, CHUNK), jnp.int32),
            pltpu.VMEM((RT, CHUNK), jnp.int32),
            pltpu.VMEM((Sp,), jnp.float32),
            pltpu.VMEM((RT, CHUNK), jnp.float32),
        ],
    )
    def zsums(sn, se, cm, srcg, dstg, sidx, oz, oew,
              snv, sev, cmv, sgv, dgv, siv, zl, eav):
        c = lax.axis_index("c")
        s = lax.axis_index("s")
        wid = c * NSUB + s
        pltpu.sync_copy(sn, snv)
        pltpu.sync_copy(se, sev)
        pltpu.sync_copy(cm, cmv)
        pltpu.sync_copy(srcg.at[pl.ds(wid * RT, RT)], sgv)
        pltpu.sync_copy(dstg.at[pl.ds(wid * RT, RT)], dgv)
        pltpu.sync_copy(sidx.at[pl.ds(wid * RT, RT)], siv)
        _zero_vec_loop(zl, Sp // LANES)
        cmax = cmv[...]

        @plsc.parallel_loop(0, RT, unroll=4)
        def _body(i):
            for g in range(CHUNK // LANES):
                dsl = pl.ds(g * LANES, LANES)
                a = plsc.load_gather(snv, [sgv[i, dsl]])
                b = plsc.load_gather(sev, [dgv[i, dsl]])
                t = a + b
                sc = jnp.where(t >= 0, t, 0.2 * t)
                mb = (a if xside else b) + cmax
                mb = jnp.where(mb >= 0, mb, 0.2 * mb)
                ew = jnp.exp(sc - mb)
                eav[i, dsl] = ew
                plsc.addupdate_scatter(zl, [siv[i, dsl]], ew)
        pltpu.sync_copy(zl, oz.at[pl.ds(wid * Sp, Sp)])
        pltpu.sync_copy(eav, oew.at[pl.ds(wid * RT, RT)])

    return zsums


def _make_segmin():
    SBLK = 64
    nblk = ROWS // SBLK

    @functools.partial(
        pl.kernel,
        mesh=_mesh(),
        compiler_params=pltpu.CompilerParams(needs_layout_passes=False),
        out_type=jax.ShapeDtypeStruct((MP_, H), jnp.float32),
        scratch_types=[
            pltpu.VMEM((64, CHUNK), jnp.int32),
            pltpu.VMEM((64, CHUNK), jnp.int32),
            pltpu.VMEM((QCAP,), jnp.int32),
            pltpu.VMEM((QCAP,), jnp.int32),
            pltpu.VMEM((CHUNK, H), jnp.float32),
            pltpu.VMEM((MRANGE + 1, H), jnp.float32),
            pltpu.SemaphoreType.DMA,
        ],
    )
    def segmin(xn3, srcg, dstm, out, sgb, dmb, qsrc, qloc, rows, acc, sem):
        c = lax.axis_index("c")
        s = lax.axis_index("s")
        wid = c * NSUB + s
        lo = wid * MRANGE

        def ini(i, _):
            acc[i // 8, pl.ds((i % 8) * LANES, LANES)] = jnp.full(
                (LANES,), jnp.inf, jnp.float32)
            return 0
        lax.fori_loop(0, (MRANGE + 1) * 8, ini, 0)

        def iniq(i, _):
            dsl = pl.ds(i * LANES, LANES)
            qsrc[dsl] = jnp.zeros((LANES,), jnp.int32)
            qloc[dsl] = jnp.full((LANES,), MRANGE, jnp.int32)
            return 0
        lax.fori_loop(0, QCAP // LANES, iniq, 0)

        def blk(b, qpos):
            pltpu.sync_copy(srcg.at[pl.ds(b * SBLK, SBLK)], sgb)
            pltpu.sync_copy(dstm.at[pl.ds(b * SBLK, SBLK)], dmb)

            def row(i, qp):
                for g in range(CHUNK // LANES):
                    dsl = pl.ds(g * LANES, LANES)
                    d16 = dmb[i, dsl]
                    s16 = sgb[i, dsl]
                    m = (d16 >= lo) & (d16 < lo + MRANGE)
                    cnt = jnp.sum(m.astype(jnp.int32))
                    qp = jnp.minimum(qp, QCAP - LANES)
                    plsc.store_compressed(qsrc.at[pl.ds(qp, LANES)], s16,
                                          mask=m)
                    plsc.store_compressed(qloc.at[pl.ds(qp, LANES)],
                                          d16 - lo, mask=m)
                    qp = qp + cnt
                return qp

            return lax.fori_loop(0, SBLK, row, qpos)

        qpos = lax.fori_loop(0, nblk, blk, 0)
        nb = (qpos + CHUNK - 1) // CHUNK

        def bat(b, _):
            pltpu.async_copy(xn3.at[qsrc.at[pl.ds(b * CHUNK, CHUNK)]],
                             rows, sem).wait()

            def per_grp(g, _):
                ql16 = qloc[pl.ds(b * CHUNK + g * LANES, LANES)]
                for lane in range(LANES):
                    e = g * LANES + lane
                    dl = ql16[lane]
                    for j in range(H // LANES):
                        dj = pl.ds(j * LANES, LANES)
                        acc[dl, dj] = jnp.minimum(acc[dl, dj], rows[e, dj])
                return 0

            lax.fori_loop(0, CHUNK // LANES, per_grp, 0)
            return 0

        lax.fori_loop(0, nb, bat, 0)
        pltpu.sync_copy(acc.at[pl.ds(0, MRANGE)], out.at[pl.ds(lo, MRANGE)])

    return segmin


def kernel(x, x_struct, x_e, edge_index, W_her, b_her, W_sfr, b_sfr, Wn1, We1, an1, ae1, Wn2, We2, an2, ae2, Wn3, We3, an3, ae3, gamma1, beta1, W_fuse, b_fuse, gamma2, beta2, W_c1, b_c1, W_c2, b_c2):
    src, dst = edge_index[0], edge_index[1]
    N = x.shape[0]
    M = x_e.shape[0]
    E = src.shape[0]
    pad = EP - E

    i32 = jnp.int32
    ew_dummy = jnp.zeros((ROWS, CHUNK), jnp.float32)
    srcg = jnp.concatenate([src, jnp.zeros((pad,), i32)]).reshape(ROWS, CHUNK)
    dstg = jnp.concatenate([dst, jnp.zeros((pad,), i32)]).reshape(ROWS, CHUNK)
    srcs = jnp.concatenate(
        [src, jnp.full((pad,), NP_ - 1, i32)]).reshape(ROWS, CHUNK)
    dsts = jnp.concatenate(
        [dst, jnp.full((pad,), MP_ - 1, i32)]).reshape(ROWS, CHUNK)
    dstm = jnp.concatenate(
        [dst, jnp.full((pad,), DSENT, i32)]).reshape(ROWS, CHUNK)

    USE_COUNTS, USE_ROWSUM, USE_LAYER, USE_MIN = True, True, True, True
    counts_k = _make_counts()
    rowsum_m = _make_rowsum(MP_, False)
    rowsum_n = _make_rowsum(NP_, False)
    wrowsum_n = _make_rowsum(NP_, True)
    wrowsum_m = _make_rowsum(MP_, True)
    zsums_x = _make_zsums(True)
    zsums_e = _make_zsums(False)
    segmin_k = _make_segmin()

    xe = _mm(x_e, W_her, b_her, act=True)
    xn0 = _mm(x, W_sfr, b_sfr, act=True)

    if USE_COUNTS:
        cs_p, cd_p = counts_k(srcs, dsts)
        cnt_src = cs_p.reshape(NW, NP_).sum(axis=0)[:N]
        cnt_dst = cd_p.reshape(NW, MP_).sum(axis=0)[:M]
    else:
        cnt_src = jax.ops.segment_sum(jnp.ones((E,)), src, num_segments=N)
        cnt_dst = jax.ops.segment_sum(jnp.ones((E,)), dst, num_segments=M)

    if USE_ROWSUM:
        he_p = rowsum_m(xn0, srcg, dsts, ew_dummy)
        he_s = (he_p[0] + he_p[1])[:M]
    else:
        he_s = jax.ops.segment_sum(xn0[src], dst, num_segments=M)
    he = he_s / jnp.maximum(cnt_dst, 1.0)[:, None]
    if USE_ROWSUM:
        bk_p = rowsum_n(he, dstg, srcs, ew_dummy)
        bk_s = (bk_p[0] + bk_p[1])[:N]
    else:
        bk_s = jax.ops.segment_sum(he[dst], src, num_segments=N)
    xn = jnp.tanh(xn0 + bk_s / jnp.maximum(cnt_src, 1.0)[:, None])

    zero_b = jnp.zeros((H,), jnp.float32)
    for Wn, We, an, ae in ((Wn1, We1, an1, ae1), (Wn2, We2, an2, ae2),
                           (Wn3, We3, an3, ae3)):
        qn = _mm(xn, Wn, zero_b, act=False)
        qe = _mm(xe, We, zero_b, act=False)
        sn = qn @ an
        se = qe @ ae
        if USE_LAYER:
            cse = jnp.full((LANES,), se.max(), jnp.float32)
            csn = jnp.full((LANES,), sn.max(), jnp.float32)
            za, ea2 = zsums_x(sn, se, cse, srcg, dstg, srcs)
            zb, eb2 = zsums_e(sn, se, csn, srcg, dstg, dsts)
            xacc = wrowsum_n(qe, dstg, srcs, ea2)
            eacc = wrowsum_m(qn, srcg, dsts, eb2)
            u = (xacc[0] + xacc[1])[:N]
            v = (eacc[0] + eacc[1])[:M]
            zan = za.reshape(NW, NP_).sum(axis=0)[:N]
            zbm = zb.reshape(NW, MP_).sum(axis=0)[:M]
        else:
            mhn = jax.nn.leaky_relu(sn + se.max(), 0.2)
            mhe = jax.nn.leaky_relu(se + sn.max(), 0.2)
            score = jax.nn.leaky_relu(sn[src] + se[dst], 0.2)
            ea = jnp.exp(score - mhn[src])
            eb = jnp.exp(score - mhe[dst])
            u = jax.ops.segment_sum(ea[:, None] * qe[dst], src, num_segments=N)
            v = jax.ops.segment_sum(eb[:, None] * qn[src], dst, num_segments=M)
            zan = jax.ops.segment_sum(ea, src, num_segments=N)
            zbm = jax.ops.segment_sum(eb, dst, num_segments=M)
        xn = jnp.tanh(u / (zan[:, None] + 1e-16) + xn)
        xe = jnp.tanh(v / (zbm[:, None] + 1e-16) + xe)

    if USE_MIN:
        agg = segmin_k(xn, srcg, dstm)[:M]
    else:
        agg = jax.ops.segment_min(xn[src], dst, num_segments=M)
    h = jnp.concatenate([agg, xe], axis=1)
    h = _bn(h, gamma1, beta1)
    h = jnp.tanh(_mm(h, W_fuse, b_fuse, act=False))
    h = _bn(h, gamma2, beta2)
    h = jnp.tanh(_mm(h, W_c1, b_c1, act=False))
    return _mm(h, W_c2, b_c2, act=False)
